# Initial kernel scaffold; baseline (speedup 1.0000x reference)
import jax
import jax.numpy as jnp
from jax.experimental import pallas as pl


def kernel(ui_mat, cf_list):
    raise NotImplementedError("write your pallas kernel here")


# speedup vs baseline: 3.7765x; 3.7765x over previous
import functools

import jax
import jax.numpy as jnp
from jax import lax
from jax.experimental import pallas as pl
from jax.experimental.pallas import tpu as pltpu
from jax.experimental.pallas import tpu_sc as plsc

_N_USERS = 8192
_N_ITEMS = 16384
_FLAT = _N_USERS * _N_ITEMS
_E = 2_000_000
_LANE = 128
_ROWS_PER_STAGE = 64
_NW = 32
_ROWS_TOTAL = 16384
_E_PAD = _ROWS_TOTAL * _LANE
_ROWS_PER_TILE = _ROWS_TOTAL // _NW
_STAGES = _ROWS_PER_TILE // _ROWS_PER_STAGE


def _zero_body(o_ref):
    o_ref[...] = jnp.zeros_like(o_ref)


_zero_call = pl.pallas_call(
    _zero_body,
    grid=(64,),
    out_specs=pl.BlockSpec((_N_USERS // 64, _N_ITEMS), lambda i: (i, 0)),
    out_shape=jax.ShapeDtypeStruct((_N_USERS, _N_ITEMS), jnp.float32),
)

_mesh = plsc.VectorSubcoreMesh(core_axis_name="c", subcore_axis_name="s")


@functools.partial(
    pl.kernel,
    mesh=_mesh,
    out_type=(),
    scratch_types=[
        pltpu.VMEM((_ROWS_PER_STAGE, _LANE), jnp.int32),
        pltpu.VMEM((_ROWS_PER_STAGE, _LANE), jnp.int32),
        pltpu.VMEM((_LANE,), jnp.float32),
        pltpu.SemaphoreType.DMA,
    ],
)
def _sc_scatter(u_hbm, v_hbm, out_ref, u_v, v_v, ones_v, sem):
    wid = lax.axis_index("c") * 16 + lax.axis_index("s")
    for i in range(_LANE // 16):
        ones_v[pl.ds(i * 16, 16)] = jnp.full((16,), 1.0, jnp.float32)

    def stage(s, carry):
        row0 = wid * _ROWS_PER_TILE + s * _ROWS_PER_STAGE
        pltpu.sync_copy(u_hbm.at[pl.ds(row0, _ROWS_PER_STAGE)], u_v)
        pltpu.sync_copy(v_hbm.at[pl.ds(row0, _ROWS_PER_STAGE)], v_v)

        def comp(j, c2):
            for i in range(_LANE // 16):
                sl = pl.ds(i * 16, 16)
                v_v[j, sl] = (u_v[j, sl] << 14) + v_v[j, sl]
            return c2

        lax.fori_loop(0, _ROWS_PER_STAGE, comp, 0)
        cps = [pltpu.async_copy(ones_v, out_ref.at[v_v.at[j]], sem)
               for j in range(_ROWS_PER_STAGE)]
        for cp in cps:
            cp.wait()
        return carry

    lax.fori_loop(0, _STAGES, stage, 0)


def kernel(ui_mat, cf_list):
    pad = _E_PAD - _E
    u = jnp.concatenate([cf_list[0], cf_list[0][:pad]]).reshape(_ROWS_TOTAL, _LANE)
    v = jnp.concatenate([cf_list[1], cf_list[1][:pad]]).reshape(_ROWS_TOTAL, _LANE)
    zeros = _zero_call().reshape(_FLAT)
    buf = jax.new_ref(zeros)
    _sc_scatter(u, v, buf)
    return jax.freeze(buf).reshape(_N_USERS, _N_ITEMS)
